# baseline (device time: 123924 ns/iter reference)
import jax
import jax.numpy as jnp
from jax import lax
from jax.experimental import pallas as pl
from jax.experimental.pallas import tpu as pltpu

N_DEV = 16
M = 1024
N = 1024
CHUNK = M // N_DEV


def kernel(dy, W):
    def body(dy_ref, w_ref, out_ref, acc_ref, rs_buf, ag_buf,
             rs_send_sems, rs_recv_sems, ag_send_sems, ag_recv_sems):
        my = lax.axis_index("i")
        left = lax.rem(my + N_DEV - 1, N_DEV)
        right = lax.rem(my + 1, N_DEV)

        barrier = pltpu.get_barrier_semaphore()
        for nbr in (left, right):
            pl.semaphore_signal(
                barrier, inc=1,
                device_id=(nbr,), device_id_type=pl.DeviceIdType.MESH,
            )
        pl.semaphore_wait(barrier, 2)

        partial = lax.dot_general(
            dy_ref[...].astype(jnp.bfloat16),
            w_ref[...].astype(jnp.bfloat16),
            (((1,), (1,)), ((), ())),
            preferred_element_type=jnp.float32,
        )
        acc_ref[...] = partial.astype(jnp.bfloat16)

        def chunk_rows(c):
            return pl.ds(c * CHUNK, CHUNK)

        for s in range(N_DEV - 1):
            send_c = lax.rem(my + 2 * N_DEV - s, N_DEV)
            recv_c = lax.rem(my + 2 * N_DEV - s - 1, N_DEV)
            rdma = pltpu.make_async_remote_copy(
                src_ref=acc_ref.at[chunk_rows(send_c), :],
                dst_ref=rs_buf.at[s],
                send_sem=rs_send_sems.at[s],
                recv_sem=rs_recv_sems.at[s],
                device_id=(right,),
                device_id_type=pl.DeviceIdType.MESH,
            )
            rdma.start()
            rdma.wait()
            acc_ref[chunk_rows(recv_c), :] = (
                acc_ref[chunk_rows(recv_c), :] + rs_buf[s]
            )

        for s in range(N_DEV - 1):
            send_c = lax.rem(my + 2 * N_DEV + 1 - s, N_DEV)
            recv_c = lax.rem(my + 2 * N_DEV - s, N_DEV)
            rdma = pltpu.make_async_remote_copy(
                src_ref=acc_ref.at[chunk_rows(send_c), :],
                dst_ref=ag_buf.at[s],
                send_sem=ag_send_sems.at[s],
                recv_sem=ag_recv_sems.at[s],
                device_id=(right,),
                device_id_type=pl.DeviceIdType.MESH,
            )
            rdma.start()
            rdma.wait()
            acc_ref[chunk_rows(recv_c), :] = ag_buf[s]

        out_ref[...] = acc_ref[...].astype(jnp.float32)

    return pl.pallas_call(
        body,
        out_shape=jax.ShapeDtypeStruct((M, N), jnp.float32),
        in_specs=[
            pl.BlockSpec(memory_space=pltpu.VMEM),
            pl.BlockSpec(memory_space=pltpu.VMEM),
        ],
        out_specs=pl.BlockSpec(memory_space=pltpu.VMEM),
        scratch_shapes=[
            pltpu.VMEM((M, N), jnp.bfloat16),
            pltpu.VMEM((N_DEV - 1, CHUNK, N), jnp.bfloat16),
            pltpu.VMEM((N_DEV - 1, CHUNK, N), jnp.bfloat16),
            pltpu.SemaphoreType.DMA((N_DEV - 1,)),
            pltpu.SemaphoreType.DMA((N_DEV - 1,)),
            pltpu.SemaphoreType.DMA((N_DEV - 1,)),
            pltpu.SemaphoreType.DMA((N_DEV - 1,)),
        ],
        compiler_params=pltpu.CompilerParams(collective_id=0),
    )(dy, W)


# device time: 73700 ns/iter; 1.6815x vs baseline; 1.6815x over previous
import functools

import jax
import jax.numpy as jnp
from jax import lax
from jax.experimental import pallas as pl
from jax.experimental.pallas import tpu as pltpu

N_DEV = 16
P = 4
M = 1024
N = 1024
RB = M // P
CH = N // 2
CQ = N // 4


def kernel(dy, W):
    def body(dy_ref, w_ref, out_ref, acc_ref,
             a_cw_buf, a_ccw_buf, b1_buf, b2_buf, b3_buf, b4_buf,
             c_cw_buf, c_ccw_buf,
             a_cw_s, a_cw_r, a_ccw_s, a_ccw_r,
             b_s, b_r,
             c_cw_s, c_cw_r, c_ccw_s, c_ccw_r):
        my = lax.axis_index("i")
        z = my // P
        w = lax.rem(my, P)
        cw = z * P + lax.rem(w + 1, P)
        ccw = z * P + lax.rem(w + 3, P)
        p1 = my ^ 4
        p2 = my ^ 8
        b0 = lax.rem(z, 2)
        b1 = lax.rem(z // 2, 2)

        peers = (cw, ccw, p1, p2)
        barrier = pltpu.get_barrier_semaphore()
        for nbr in peers:
            pl.semaphore_signal(
                barrier, inc=1,
                device_id=(nbr,), device_id_type=pl.DeviceIdType.MESH,
            )
        pl.semaphore_wait(barrier, len(peers))

        partial = lax.dot_general(
            dy_ref[...].astype(jnp.bfloat16),
            w_ref[...].astype(jnp.bfloat16),
            (((1,), (1,)), ((), ())),
            preferred_element_type=jnp.float32,
        )
        acc_ref[...] = partial.astype(jnp.bfloat16)

        def rows_of(blk):
            return pl.ds(blk * RB, RB)

        def copy(src, dst, send_sem, recv_sem, dev):
            return pltpu.make_async_remote_copy(
                src_ref=src, dst_ref=dst, send_sem=send_sem,
                recv_sem=recv_sem, device_id=(dev,),
                device_id_type=pl.DeviceIdType.MESH,
            )

        for s in range(P - 1):
            sc = lax.rem(w - s + 2 * P, P)
            rc = lax.rem(w - s - 1 + 2 * P, P)
            sj = lax.rem(w + s, P)
            rj = lax.rem(w + s + 1, P)
            r_cw = copy(acc_ref.at[rows_of(sc), pl.ds(0, CH)],
                        a_cw_buf.at[s], a_cw_s.at[s], a_cw_r.at[s], cw)
            r_ccw = copy(acc_ref.at[rows_of(lax.rem(sj + 2, P)), pl.ds(CH, CH)],
                         a_ccw_buf.at[s], a_ccw_s.at[s], a_ccw_r.at[s], ccw)
            r_cw.start()
            r_ccw.start()
            r_cw.wait()
            r_ccw.wait()
            acc_ref[rows_of(rc), pl.ds(0, CH)] = (
                acc_ref[rows_of(rc), pl.ds(0, CH)] + a_cw_buf[s]
            )
            rj2 = lax.rem(rj + 2, P)
            acc_ref[rows_of(rj2), pl.ds(CH, CH)] = (
                acc_ref[rows_of(rj2), pl.ds(CH, CH)] + a_ccw_buf[s]
            )

        R = lax.rem(w + 1, P)
        rows = rows_of(R)
        half_keep = b0 * CH
        half_send = (1 - b0) * CH
        q_keep = half_keep + b1 * CQ
        q_send = half_keep + (1 - b1) * CQ

        r = copy(acc_ref.at[rows, pl.ds(half_send, CH)], b1_buf,
                 b_s.at[0], b_r.at[0], p1)
        r.start()
        r.wait()
        acc_ref[rows, pl.ds(half_keep, CH)] = (
            acc_ref[rows, pl.ds(half_keep, CH)] + b1_buf[...]
        )
        r = copy(acc_ref.at[rows, pl.ds(q_send, CQ)], b2_buf,
                 b_s.at[1], b_r.at[1], p2)
        r.start()
        r.wait()
        acc_ref[rows, pl.ds(q_keep, CQ)] = (
            acc_ref[rows, pl.ds(q_keep, CQ)] + b2_buf[...]
        )
        r = copy(acc_ref.at[rows, pl.ds(q_keep, CQ)], b3_buf,
                 b_s.at[2], b_r.at[2], p2)
        r.start()
        r.wait()
        acc_ref[rows, pl.ds(q_send, CQ)] = b3_buf[...]
        r = copy(acc_ref.at[rows, pl.ds(half_keep, CH)], b4_buf,
                 b_s.at[3], b_r.at[3], p1)
        r.start()
        r.wait()
        acc_ref[rows, pl.ds(half_send, CH)] = b4_buf[...]

        for s in range(P - 1):
            sc = lax.rem(w + 1 - s + 2 * P, P)
            rc = lax.rem(w - s + 2 * P, P)
            sj = lax.rem(w - 1 + s + 2 * P, P)
            rj = lax.rem(w + s, P)
            r_cw = copy(acc_ref.at[rows_of(sc), pl.ds(0, CH)],
                        c_cw_buf.at[s], c_cw_s.at[s], c_cw_r.at[s], cw)
            r_ccw = copy(acc_ref.at[rows_of(lax.rem(sj + 2, P)), pl.ds(CH, CH)],
                         c_ccw_buf.at[s], c_ccw_s.at[s], c_ccw_r.at[s], ccw)
            r_cw.start()
            r_ccw.start()
            r_cw.wait()
            r_ccw.wait()
            acc_ref[rows_of(rc), pl.ds(0, CH)] = c_cw_buf[s]
            rj2 = lax.rem(rj + 2, P)
            acc_ref[rows_of(rj2), pl.ds(CH, CH)] = c_ccw_buf[s]

        out_ref[...] = acc_ref[...].astype(jnp.float32)

        @functools.partial(
            pl.run_scoped, second_barrier=pltpu.SemaphoreType.REGULAR
        )
        def _(second_barrier):
            for nbr in peers:
                pl.semaphore_signal(
                    second_barrier, inc=1,
                    device_id=(nbr,), device_id_type=pl.DeviceIdType.MESH,
                )
            pl.semaphore_wait(second_barrier, len(peers))

    return pl.pallas_call(
        body,
        out_shape=jax.ShapeDtypeStruct((M, N), jnp.float32),
        in_specs=[
            pl.BlockSpec(memory_space=pltpu.VMEM),
            pl.BlockSpec(memory_space=pltpu.VMEM),
        ],
        out_specs=pl.BlockSpec(memory_space=pltpu.VMEM),
        scratch_shapes=[
            pltpu.VMEM((M, N), jnp.bfloat16),
            pltpu.VMEM((P - 1, RB, CH), jnp.bfloat16),
            pltpu.VMEM((P - 1, RB, CH), jnp.bfloat16),
            pltpu.VMEM((RB, CH), jnp.bfloat16),
            pltpu.VMEM((RB, CQ), jnp.bfloat16),
            pltpu.VMEM((RB, CQ), jnp.bfloat16),
            pltpu.VMEM((RB, CH), jnp.bfloat16),
            pltpu.VMEM((P - 1, RB, CH), jnp.bfloat16),
            pltpu.VMEM((P - 1, RB, CH), jnp.bfloat16),
            pltpu.SemaphoreType.DMA((P - 1,)),
            pltpu.SemaphoreType.DMA((P - 1,)),
            pltpu.SemaphoreType.DMA((P - 1,)),
            pltpu.SemaphoreType.DMA((P - 1,)),
            pltpu.SemaphoreType.DMA((4,)),
            pltpu.SemaphoreType.DMA((4,)),
            pltpu.SemaphoreType.DMA((P - 1,)),
            pltpu.SemaphoreType.DMA((P - 1,)),
            pltpu.SemaphoreType.DMA((P - 1,)),
            pltpu.SemaphoreType.DMA((P - 1,)),
        ],
        compiler_params=pltpu.CompilerParams(collective_id=0),
    )(dy, W)


# device time: 69352 ns/iter; 1.7869x vs baseline; 1.0627x over previous
import functools

import jax
import jax.numpy as jnp
from jax import lax
from jax.experimental import pallas as pl
from jax.experimental.pallas import tpu as pltpu

N_DEV = 16
P = 4
M = 1024
N = 1024
RB = M // P
CH = N // 2
CQ = N // 4


def kernel(dy, W):
    def body(dy_ref, w_ref, out_ref, acc_ref,
             a_cw_buf, a_ccw_buf, b1_buf, b2_buf, b3_buf, b4_buf,
             c_cw_buf, c_ccw_buf,
             a_cw_s, a_cw_r, a_ccw_s, a_ccw_r,
             b_s, b_r,
             c_cw_s, c_cw_r, c_ccw_s, c_ccw_r):
        my = lax.axis_index("i")
        z = my // P
        w = lax.rem(my, P)
        cw = z * P + lax.rem(w + 1, P)
        ccw = z * P + lax.rem(w + 3, P)
        p1 = my ^ 4
        p2 = my ^ 8
        b0 = lax.rem(z, 2)
        b1 = lax.rem(z // 2, 2)

        peers = (cw, ccw, p1, p2)
        barrier = pltpu.get_barrier_semaphore()
        for nbr in peers:
            pl.semaphore_signal(
                barrier, inc=1,
                device_id=(nbr,), device_id_type=pl.DeviceIdType.MESH,
            )
        pl.semaphore_wait(barrier, len(peers))

        def rows_of(blk):
            return pl.ds(blk * RB, RB)

        def copy(src, dst, send_sem, recv_sem, dev):
            return pltpu.make_async_remote_copy(
                src_ref=src, dst_ref=dst, send_sem=send_sem,
                recv_sem=recv_sem, device_id=(dev,),
                device_id_type=pl.DeviceIdType.MESH,
            )

        pending = []

        def gemm_block(blk):
            acc_ref[rows_of(blk), :] = lax.dot_general(
                dy_ref[rows_of(blk), :].astype(jnp.bfloat16),
                w_ref[...].astype(jnp.bfloat16),
                (((1,), (1,)), ((), ())),
                preferred_element_type=jnp.float32,
            ).astype(jnp.bfloat16)

        def a_step_rdmas(s):
            sc = lax.rem(w - s + 2 * P, P)
            sj = lax.rem(w + s, P)
            r_cw = copy(acc_ref.at[rows_of(sc), pl.ds(0, CH)],
                        a_cw_buf.at[s], a_cw_s.at[s], a_cw_r.at[s], cw)
            r_ccw = copy(acc_ref.at[rows_of(lax.rem(sj + 2, P)), pl.ds(CH, CH)],
                         a_ccw_buf.at[s], a_ccw_s.at[s], a_ccw_r.at[s], ccw)
            r_cw.start()
            r_ccw.start()
            pending.extend((r_cw, r_ccw))
            return r_cw, r_ccw

        gemm_block(w)
        gemm_block(lax.rem(w + 2, P))
        a0 = a_step_rdmas(0)
        gemm_block(lax.rem(w + 3, P))
        gemm_block(lax.rem(w + 1, P))

        rdmas = a0
        for s in range(P - 1):
            rc = lax.rem(w - s - 1 + 2 * P, P)
            rj = lax.rem(w + s + 1, P)
            rdmas[0].wait_recv()
            rdmas[1].wait_recv()
            acc_ref[rows_of(rc), pl.ds(0, CH)] = (
                acc_ref[rows_of(rc), pl.ds(0, CH)] + a_cw_buf[s]
            )
            rj2 = lax.rem(rj + 2, P)
            acc_ref[rows_of(rj2), pl.ds(CH, CH)] = (
                acc_ref[rows_of(rj2), pl.ds(CH, CH)] + a_ccw_buf[s]
            )
            if s < P - 2:
                rdmas = a_step_rdmas(s + 1)

        R = lax.rem(w + 1, P)
        rows = rows_of(R)
        half_keep = b0 * CH
        half_send = (1 - b0) * CH
        q_keep = half_keep + b1 * CQ
        q_send = half_keep + (1 - b1) * CQ

        r = copy(acc_ref.at[rows, pl.ds(half_send, CH)], b1_buf,
                 b_s.at[0], b_r.at[0], p1)
        r.start()
        pending.append(r)
        r.wait_recv()
        acc_ref[rows, pl.ds(half_keep, CH)] = (
            acc_ref[rows, pl.ds(half_keep, CH)] + b1_buf[...]
        )
        r = copy(acc_ref.at[rows, pl.ds(q_send, CQ)], b2_buf,
                 b_s.at[1], b_r.at[1], p2)
        r.start()
        pending.append(r)
        r.wait_recv()
        acc_ref[rows, pl.ds(q_keep, CQ)] = (
            acc_ref[rows, pl.ds(q_keep, CQ)] + b2_buf[...]
        )
        r = copy(acc_ref.at[rows, pl.ds(q_keep, CQ)], b3_buf,
                 b_s.at[2], b_r.at[2], p2)
        r.start()
        pending.append(r)
        r.wait_recv()
        acc_ref[rows, pl.ds(q_send, CQ)] = b3_buf[...]
        r = copy(acc_ref.at[rows, pl.ds(half_keep, CH)], b4_buf,
                 b_s.at[3], b_r.at[3], p1)
        r.start()
        pending.append(r)
        r.wait_recv()
        acc_ref[rows, pl.ds(half_send, CH)] = b4_buf[...]
        out_ref[rows, :] = acc_ref[rows, :].astype(jnp.float32)

        def c_step_rdmas(s):
            if s == 0:
                src_cw = acc_ref.at[rows, pl.ds(0, CH)]
                src_ccw = acc_ref.at[rows, pl.ds(CH, CH)]
            else:
                src_cw = c_cw_buf.at[s - 1]
                src_ccw = c_ccw_buf.at[s - 1]
            r_cw = copy(src_cw, c_cw_buf.at[s],
                        c_cw_s.at[s], c_cw_r.at[s], cw)
            r_ccw = copy(src_ccw, c_ccw_buf.at[s],
                         c_ccw_s.at[s], c_ccw_r.at[s], ccw)
            r_cw.start()
            r_ccw.start()
            pending.extend((r_cw, r_ccw))
            return r_cw, r_ccw

        rdmas = c_step_rdmas(0)
        for s in range(P - 1):
            rc = lax.rem(w - s + 2 * P, P)
            rj = lax.rem(w + s, P)
            rdmas[0].wait_recv()
            rdmas[1].wait_recv()
            if s < P - 2:
                rdmas = c_step_rdmas(s + 1)
            out_ref[rows_of(rc), pl.ds(0, CH)] = (
                c_cw_buf[s].astype(jnp.float32)
            )
            rj2 = lax.rem(rj + 2, P)
            out_ref[rows_of(rj2), pl.ds(CH, CH)] = (
                c_ccw_buf[s].astype(jnp.float32)
            )

        for r in pending:
            r.wait_send()

        @functools.partial(
            pl.run_scoped, second_barrier=pltpu.SemaphoreType.REGULAR
        )
        def _(second_barrier):
            for nbr in peers:
                pl.semaphore_signal(
                    second_barrier, inc=1,
                    device_id=(nbr,), device_id_type=pl.DeviceIdType.MESH,
                )
            pl.semaphore_wait(second_barrier, len(peers))

    return pl.pallas_call(
        body,
        out_shape=jax.ShapeDtypeStruct((M, N), jnp.float32),
        in_specs=[
            pl.BlockSpec(memory_space=pltpu.VMEM),
            pl.BlockSpec(memory_space=pltpu.VMEM),
        ],
        out_specs=pl.BlockSpec(memory_space=pltpu.VMEM),
        scratch_shapes=[
            pltpu.VMEM((M, N), jnp.bfloat16),
            pltpu.VMEM((P - 1, RB, CH), jnp.bfloat16),
            pltpu.VMEM((P - 1, RB, CH), jnp.bfloat16),
            pltpu.VMEM((RB, CH), jnp.bfloat16),
            pltpu.VMEM((RB, CQ), jnp.bfloat16),
            pltpu.VMEM((RB, CQ), jnp.bfloat16),
            pltpu.VMEM((RB, CH), jnp.bfloat16),
            pltpu.VMEM((P - 1, RB, CH), jnp.bfloat16),
            pltpu.VMEM((P - 1, RB, CH), jnp.bfloat16),
            pltpu.SemaphoreType.DMA((P - 1,)),
            pltpu.SemaphoreType.DMA((P - 1,)),
            pltpu.SemaphoreType.DMA((P - 1,)),
            pltpu.SemaphoreType.DMA((P - 1,)),
            pltpu.SemaphoreType.DMA((4,)),
            pltpu.SemaphoreType.DMA((4,)),
            pltpu.SemaphoreType.DMA((P - 1,)),
            pltpu.SemaphoreType.DMA((P - 1,)),
            pltpu.SemaphoreType.DMA((P - 1,)),
            pltpu.SemaphoreType.DMA((P - 1,)),
        ],
        compiler_params=pltpu.CompilerParams(collective_id=0),
    )(dy, W)
